# baseline (speedup 1.0000x reference)
import functools

import jax
import jax.numpy as jnp
from jax import lax
from jax.experimental import pallas as pl
from jax.experimental.pallas import tpu as pltpu
from jax.experimental.pallas import tpu_sc as plsc

NC = 2
NS = 16
L = 16

HALF = 64


def _tables_body(x_ref, b_ref, d_ref, ew_ref, tsrc_ref, tdst_ref):
    xb = x_ref[...]
    xB = jnp.dot(xb, b_ref[...], preferred_element_type=jnp.float32)
    xD = jnp.dot(xb, d_ref[...], preferred_element_type=jnp.float32)
    xE = jnp.dot(xb, ew_ref[...], preferred_element_type=jnp.float32)
    tsrc_ref[0] = jnp.concatenate([xE[:, :HALF], xB[:, :HALF]], axis=1)
    tsrc_ref[1] = jnp.concatenate([xE[:, HALF:], xB[:, HALF:]], axis=1)
    z = jnp.zeros_like(xD[:, :HALF])
    tdst_ref[0] = jnp.concatenate([xD[:, :HALF], z], axis=1)
    tdst_ref[1] = jnp.concatenate([xD[:, HALF:], z], axis=1)


def _make_tables(x, B, D, Ew):
    N, d = x.shape
    NB = 2000
    grid = (N // NB,)
    return pl.pallas_call(
        _tables_body,
        grid=grid,
        in_specs=[
            pl.BlockSpec((NB, d), lambda i: (i, 0)),
            pl.BlockSpec((d, d), lambda i: (0, 0)),
            pl.BlockSpec((d, d), lambda i: (0, 0)),
            pl.BlockSpec((d, d), lambda i: (0, 0)),
        ],
        out_specs=[
            pl.BlockSpec((2, NB, d), lambda i: (0, i, 0)),
            pl.BlockSpec((2, NB, d), lambda i: (0, i, 0)),
        ],
        out_shape=[
            jax.ShapeDtypeStruct((2, N, d), jnp.float32),
            jax.ShapeDtypeStruct((2, N, d), jnp.float32),
        ],
    )(x, B, D, Ew)


def _ec_body(ea_ref, c_ref, out_ref):
    ev = jnp.dot(ea_ref[...], c_ref[...], preferred_element_type=jnp.float32)
    out_ref[0] = ev[:, :HALF]
    out_ref[1] = ev[:, HALF:]


def _make_ec(edge_attr, C):
    E, de = edge_attr.shape
    d = C.shape[1]
    EB = 8000
    return pl.pallas_call(
        _ec_body,
        grid=(E // EB,),
        in_specs=[
            pl.BlockSpec((EB, de), lambda i: (i, 0)),
            pl.BlockSpec((de, d), lambda i: (0, 0)),
        ],
        out_specs=pl.BlockSpec((2, EB, HALF), lambda i: (0, i, 0)),
        out_shape=jax.ShapeDtypeStruct((2, E, HALF), jnp.float32),
    )(edge_attr, C)


def _sc_gate(N, E, BLK):
    EPT = E // NS
    NBLK = EPT // BLK
    d = 2 * HALF

    N1 = 5000
    N2 = N - N1
    AR = 5040
    ZCH = 48
    FCH = 40

    mesh = plsc.VectorSubcoreMesh(core_axis_name="c", subcore_axis_name="s")

    @functools.partial(
        pl.kernel,
        mesh=mesh,
        out_type=[
            jax.ShapeDtypeStruct((NC * N, d), jnp.float32),
            jax.ShapeDtypeStruct((NC * E, d), jnp.float32),
        ],
        scratch_types=(
            [pltpu.VMEM((BLK,), jnp.int32)] * 10 +
            [pltpu.VMEM((BLK, HALF), jnp.float32)] * 2 +
            [pltpu.VMEM((BLK, d), jnp.float32)] * 6 +
            [pltpu.VMEM((ZCH, d), jnp.float32),
             pltpu.VMEM_SHARED((AR, d), jnp.float32)] +
            [pltpu.SemaphoreType.DMA] * 4
        ),
    )
    def k(src_hbm, dst_hbm, tsrc_hbm, tdst_hbm, ec_hbm, out_hbm, spill_hbm,
          sidx0, didx0, gsidx0, gdidx0, qidx0,
          sidx1, didx1, gsidx1, gdidx1, qidx1,
          ecv0, ecv1, tsv0, tdv0, msd0, tsv1, tdv1, msd1, zbuf, acc,
          semi, semg, semsA, semsB):
        c = lax.axis_index("c")
        s = lax.axis_index("s")
        coff = c * N
        slot0 = (sidx0, didx0, gsidx0, gdidx0, qidx0, ecv0, tsv0, tdv0,
                 msd0, semsA)
        slot1 = (sidx1, didx1, gsidx1, gdidx1, qidx1, ecv1, tsv1, tdv1,
                 msd1, semsB)

        zv = jnp.zeros((L,), jnp.float32)

        def zloop(t, _):
            i = t // (d // L)
            j = t % (d // L)
            zbuf[i, pl.ds(j * L, L)] = zv
            return 0

        lax.fori_loop(0, ZCH * (d // L), zloop, 0)

        def zero_acc():
            for kk in range(-(-(AR // ZCH) // NS)):
                ch = s + kk * NS
                @pl.when(ch < AR // ZCH)
                def _():
                    pltpu.sync_copy(zbuf, acc.at[pl.ds(ch * ZCH, ZCH)])

        zero_acc()
        plsc.subcore_barrier()

        def ids_fire(blk, sl_):
            base = s * EPT + blk * BLK
            pltpu.async_copy(src_hbm.at[pl.ds(base, BLK)], sl_[0], semi)
            pltpu.async_copy(dst_hbm.at[pl.ds(base, BLK)], sl_[1], semi)

        def ids_wait(blk, sl_):
            base = s * EPT + blk * BLK
            pltpu.make_async_copy(src_hbm.at[pl.ds(base, BLK)], sl_[0],
                                  semi).wait()
            pltpu.make_async_copy(dst_hbm.at[pl.ds(base, BLK)], sl_[1],
                                  semi).wait()

        def prep(sl_):
            sb, db, gs, gd, qi = sl_[0], sl_[1], sl_[2], sl_[3], sl_[4]
            for kk in range(BLK // L):
                sl = pl.ds(kk * L, L)
                dv = db[sl]
                gs[sl] = sb[sl] + coff
                gd[sl] = dv + coff
                qi[sl] = jnp.where(dv < N1, dv, N1)

        def gath_fire(blk, sl_):
            base = s * EPT + blk * BLK
            pltpu.async_copy(tsrc_hbm.at[sl_[2]], sl_[6], semg)
            pltpu.async_copy(tdst_hbm.at[sl_[3]], sl_[7], semg)
            pltpu.async_copy(ec_hbm.at[pl.ds(c * E + base, BLK)], sl_[5],
                             semg)

        def gath_wait(blk, sl_):
            base = s * EPT + blk * BLK
            pltpu.make_async_copy(tsrc_hbm.at[sl_[2]], sl_[6], semg).wait()
            pltpu.make_async_copy(tdst_hbm.at[sl_[3]], sl_[7], semg).wait()
            pltpu.make_async_copy(ec_hbm.at[pl.ds(c * E + base, BLK)],
                                  sl_[5], semg).wait()

        def compute(sl_):
            ecb, tsb, tdb, msb = sl_[5], sl_[6], sl_[7], sl_[8]

            @functools.partial(plsc.parallel_loop, 0, BLK, unroll=2)
            def edge(j):
                for q in range(HALF // L):
                    sl = pl.ds(q * L, L)
                    sh = pl.ds(HALF + q * L, L)
                    e = ecb[j, sl] + tdb[j, sl] + tsb[j, sl]
                    sg = 1.0 / (1.0 + jnp.exp(-e))
                    msb[j, sl] = sg * tsb[j, sh]
                    msb[j, sh] = sg

        def spill_fire(blk, sl_):
            base = s * EPT + blk * BLK
            pltpu.async_copy(sl_[8],
                             spill_hbm.at[pl.ds(c * E + base, BLK)], sl_[9])

        def spill_wait(blk, sl_):
            base = s * EPT + blk * BLK
            pltpu.make_async_copy(sl_[8],
                                  spill_hbm.at[pl.ds(c * E + base, BLK)],
                                  sl_[9]).wait()

        def step1(blk, cur, nxt, fire_next, fire_ids, drain_spill):
            gath_wait(blk, cur)
            if fire_next:
                ids_wait(blk + 1, nxt)
                prep(nxt)
                gath_fire(blk + 1, nxt)
            if fire_ids:
                ids_fire(blk + 2, cur)
            if drain_spill:
                spill_wait(blk - 2, cur)
            compute(cur)
            pltpu.sync_copy(cur[8], acc.at[cur[4]], add=True)
            spill_fire(blk, cur)

        ids_fire(0, slot0)
        ids_wait(0, slot0)
        prep(slot0)
        gath_fire(0, slot0)
        ids_fire(1, slot1)

        step1(0, slot0, slot1, True, True, False)
        step1(1, slot1, slot0, True, True, False)

        def pair1(t, _):
            b0 = 2 * t
            step1(b0, slot0, slot1, True, True, True)
            step1(b0 + 1, slot1, slot0, True, True, True)
            return 0

        lax.fori_loop(1, NBLK // 2 - 1, pair1, 0)
        step1(NBLK - 2, slot0, slot1, True, False, True)
        step1(NBLK - 1, slot1, slot0, False, False, True)
        spill_wait(NBLK - 2, slot0)
        spill_wait(NBLK - 1, slot1)

        plsc.subcore_barrier()
        for kk in range(-(-(N1 // FCH) // NS)):
            ch = s + kk * NS
            @pl.when(ch < N1 // FCH)
            def _():
                pltpu.sync_copy(acc.at[pl.ds(ch * FCH, FCH)],
                                out_hbm.at[pl.ds(coff + ch * FCH, FCH)])
        plsc.subcore_barrier()

        zero_acc()
        plsc.subcore_barrier()

        def rd_fire(blk, sl_):
            base = s * EPT + blk * BLK
            pltpu.async_copy(dst_hbm.at[pl.ds(base, BLK)], sl_[1], semi)
            pltpu.async_copy(spill_hbm.at[pl.ds(c * E + base, BLK)],
                             sl_[8], semg)

        def rd_wait(blk, sl_):
            base = s * EPT + blk * BLK
            pltpu.make_async_copy(dst_hbm.at[pl.ds(base, BLK)], sl_[1],
                                  semi).wait()
            pltpu.make_async_copy(spill_hbm.at[pl.ds(c * E + base, BLK)],
                                  sl_[8], semg).wait()

        def step2(blk, cur, nxt, fire_next):
            rd_wait(blk, cur)
            if fire_next:
                rd_fire(blk + 1, nxt)
            db, qi = cur[1], cur[4]
            for kk in range(BLK // L):
                sl = pl.ds(kk * L, L)
                dv = db[sl] - N1
                qi[sl] = jnp.where(dv >= 0, dv, N2)
            pltpu.sync_copy(cur[8], acc.at[qi], add=True)

        rd_fire(0, slot0)

        def pair2(t, _):
            step2(2 * t, slot0, slot1, True)
            step2(2 * t + 1, slot1, slot0, True)
            return 0

        lax.fori_loop(0, NBLK // 2 - 1, pair2, 0)
        step2(NBLK - 2, slot0, slot1, True)
        step2(NBLK - 1, slot1, slot0, False)

        plsc.subcore_barrier()
        for kk in range(-(-(N2 // FCH) // NS)):
            ch = s + kk * NS
            @pl.when(ch < N2 // FCH)
            def _():
                pltpu.sync_copy(acc.at[pl.ds(ch * FCH, FCH)],
                                out_hbm.at[pl.ds(coff + N1 + ch * FCH, FCH)])

    return k


def _epi_body(x_ref, a_ref, a0_ref, a1_ref, out_ref):
    xb = x_ref[...]
    a0 = a0_ref[...]
    a1 = a1_ref[...]
    num = jnp.concatenate([a0[:, :HALF], a1[:, :HALF]], axis=1)
    den = jnp.concatenate([a0[:, HALF:], a1[:, HALF:]], axis=1) + 1e-6
    h = jnp.dot(xb, a_ref[...], preferred_element_type=jnp.float32)
    h = h + num / den
    out_ref[...] = xb + jnp.maximum(h, 0.0)


def _epilogue(x, A, accd):
    N, d = x.shape
    NB = 2000
    nb = N // NB
    return pl.pallas_call(
        _epi_body,
        grid=(nb,),
        in_specs=[
            pl.BlockSpec((NB, d), lambda i: (i, 0)),
            pl.BlockSpec((d, d), lambda i: (0, 0)),
            pl.BlockSpec((NB, d), lambda i: (i, 0)),
            pl.BlockSpec((NB, d), lambda i, nb=nb: (i + nb, 0)),
        ],
        out_specs=pl.BlockSpec((NB, d), lambda i: (i, 0)),
        out_shape=jax.ShapeDtypeStruct((N, d), jnp.float32),
    )(x, A, accd, accd)


def kernel(x, edge_index, edge_attr, A, B, C, D, Ew):
    N, d = x.shape
    E = edge_index.shape[1]
    src = edge_index[0]
    dst = edge_index[1]

    tsrc, tdst = _make_tables(x, B, D, Ew)
    ec = _make_ec(edge_attr, C)
    tsrc = tsrc.reshape(NC * N, d)
    tdst = tdst.reshape(NC * N, d)
    ec = ec.reshape(NC * E, HALF)

    accd, _spill = _sc_gate(N, E, BLK=80)(src, dst, tsrc, tdst, ec)
    return _epilogue(x, A, accd)


# speedup vs baseline: 1.0047x; 1.0047x over previous
import functools

import jax
import jax.numpy as jnp
from jax import lax
from jax.experimental import pallas as pl
from jax.experimental.pallas import tpu as pltpu
from jax.experimental.pallas import tpu_sc as plsc

NC = 2
NS = 16
L = 16

HALF = 64


def _tables_body(x_ref, b_ref, d_ref, ew_ref, tsrc_ref, tdst_ref):
    xb = x_ref[...]
    xB = jnp.dot(xb, b_ref[...], preferred_element_type=jnp.float32)
    xD = jnp.dot(xb, d_ref[...], preferred_element_type=jnp.float32)
    xE = jnp.dot(xb, ew_ref[...], preferred_element_type=jnp.float32)
    tsrc_ref[0] = jnp.concatenate([xE[:, :HALF], xB[:, :HALF]], axis=1)
    tsrc_ref[1] = jnp.concatenate([xE[:, HALF:], xB[:, HALF:]], axis=1)
    z = jnp.zeros_like(xD[:, :HALF])
    tdst_ref[0] = jnp.concatenate([xD[:, :HALF], z], axis=1)
    tdst_ref[1] = jnp.concatenate([xD[:, HALF:], z], axis=1)


def _make_tables(x, B, D, Ew):
    N, d = x.shape
    NB = 2000
    grid = (N // NB,)
    return pl.pallas_call(
        _tables_body,
        grid=grid,
        in_specs=[
            pl.BlockSpec((NB, d), lambda i: (i, 0)),
            pl.BlockSpec((d, d), lambda i: (0, 0)),
            pl.BlockSpec((d, d), lambda i: (0, 0)),
            pl.BlockSpec((d, d), lambda i: (0, 0)),
        ],
        out_specs=[
            pl.BlockSpec((2, NB, d), lambda i: (0, i, 0)),
            pl.BlockSpec((2, NB, d), lambda i: (0, i, 0)),
        ],
        out_shape=[
            jax.ShapeDtypeStruct((2, N, d), jnp.float32),
            jax.ShapeDtypeStruct((2, N, d), jnp.float32),
        ],
    )(x, B, D, Ew)


def _ec_body(ea_ref, c_ref, out_ref):
    ev = jnp.dot(ea_ref[...], c_ref[...], preferred_element_type=jnp.float32)
    out_ref[0] = ev[:, :HALF]
    out_ref[1] = ev[:, HALF:]


def _make_ec(edge_attr, C):
    E, de = edge_attr.shape
    d = C.shape[1]
    EB = 8000
    return pl.pallas_call(
        _ec_body,
        grid=(E // EB,),
        in_specs=[
            pl.BlockSpec((EB, de), lambda i: (i, 0)),
            pl.BlockSpec((de, d), lambda i: (0, 0)),
        ],
        out_specs=pl.BlockSpec((2, EB, HALF), lambda i: (0, i, 0)),
        out_shape=jax.ShapeDtypeStruct((2, E, HALF), jnp.float32),
    )(edge_attr, C)


def _sc_gate(N, E, BLK):
    EPT = E // NS
    NBLK = EPT // BLK
    d = 2 * HALF

    N1 = 5000
    N2 = N - N1
    AR = 5040
    ZCH = 48
    FCH = 40

    mesh = plsc.VectorSubcoreMesh(core_axis_name="c", subcore_axis_name="s")

    @functools.partial(
        pl.kernel,
        mesh=mesh,
        out_type=[
            jax.ShapeDtypeStruct((NC * N, d), jnp.float32),
            jax.ShapeDtypeStruct((NC * E, d), jnp.float32),
        ],
        scratch_types=(
            [pltpu.VMEM((BLK,), jnp.int32)] * 10 +
            [pltpu.VMEM((BLK, HALF), jnp.float32)] * 2 +
            [pltpu.VMEM((BLK, d), jnp.float32)] * 6 +
            [pltpu.VMEM((ZCH, d), jnp.float32),
             pltpu.VMEM_SHARED((AR, d), jnp.float32)] +
            [pltpu.SemaphoreType.DMA] * 6
        ),
    )
    def k(src_hbm, dst_hbm, tsrc_hbm, tdst_hbm, ec_hbm, out_hbm, spill_hbm,
          sidx0, didx0, gsidx0, gdidx0, qidx0,
          sidx1, didx1, gsidx1, gdidx1, qidx1,
          ecv0, ecv1, tsv0, tdv0, msd0, tsv1, tdv1, msd1, zbuf, acc,
          semi, semg, semsA, semsB, semcA, semcB):
        c = lax.axis_index("c")
        s = lax.axis_index("s")
        coff = c * N
        slot0 = (sidx0, didx0, gsidx0, gdidx0, qidx0, ecv0, tsv0, tdv0,
                 msd0, semsA, semcA)
        slot1 = (sidx1, didx1, gsidx1, gdidx1, qidx1, ecv1, tsv1, tdv1,
                 msd1, semsB, semcB)

        zv = jnp.zeros((L,), jnp.float32)

        def zloop(t, _):
            i = t // (d // L)
            j = t % (d // L)
            zbuf[i, pl.ds(j * L, L)] = zv
            return 0

        lax.fori_loop(0, ZCH * (d // L), zloop, 0)

        def zero_acc():
            for kk in range(-(-(AR // ZCH) // NS)):
                ch = s + kk * NS
                @pl.when(ch < AR // ZCH)
                def _():
                    pltpu.sync_copy(zbuf, acc.at[pl.ds(ch * ZCH, ZCH)])

        zero_acc()
        plsc.subcore_barrier()

        def ids_fire(blk, sl_):
            base = s * EPT + blk * BLK
            pltpu.async_copy(src_hbm.at[pl.ds(base, BLK)], sl_[0], semi)
            pltpu.async_copy(dst_hbm.at[pl.ds(base, BLK)], sl_[1], semi)

        def ids_wait(blk, sl_):
            base = s * EPT + blk * BLK
            pltpu.make_async_copy(src_hbm.at[pl.ds(base, BLK)], sl_[0],
                                  semi).wait()
            pltpu.make_async_copy(dst_hbm.at[pl.ds(base, BLK)], sl_[1],
                                  semi).wait()

        def prep(sl_):
            sb, db, gs, gd, qi = sl_[0], sl_[1], sl_[2], sl_[3], sl_[4]
            for kk in range(BLK // L):
                sl = pl.ds(kk * L, L)
                dv = db[sl]
                gs[sl] = sb[sl] + coff
                gd[sl] = dv + coff
                qi[sl] = jnp.where(dv < N1, dv, N1)

        def gath_fire(blk, sl_):
            base = s * EPT + blk * BLK
            pltpu.async_copy(tsrc_hbm.at[sl_[2]], sl_[6], semg)
            pltpu.async_copy(tdst_hbm.at[sl_[3]], sl_[7], semg)
            pltpu.async_copy(ec_hbm.at[pl.ds(c * E + base, BLK)], sl_[5],
                             semg)

        def gath_wait(blk, sl_):
            base = s * EPT + blk * BLK
            pltpu.make_async_copy(tsrc_hbm.at[sl_[2]], sl_[6], semg).wait()
            pltpu.make_async_copy(tdst_hbm.at[sl_[3]], sl_[7], semg).wait()
            pltpu.make_async_copy(ec_hbm.at[pl.ds(c * E + base, BLK)],
                                  sl_[5], semg).wait()

        def compute(sl_):
            ecb, tsb, tdb, msb = sl_[5], sl_[6], sl_[7], sl_[8]

            def edge(j, _):
                for q in range(HALF // L):
                    sl = pl.ds(q * L, L)
                    sh = pl.ds(HALF + q * L, L)
                    e = ecb[j, sl] + tdb[j, sl] + tsb[j, sl]
                    sg = 1.0 / (1.0 + jnp.exp(-e))
                    msb[j, sl] = sg * tsb[j, sh]
                    msb[j, sh] = sg
                return 0

            lax.fori_loop(0, BLK, edge, 0)

        def spill_fire(blk, sl_):
            base = s * EPT + blk * BLK
            pltpu.async_copy(sl_[8],
                             spill_hbm.at[pl.ds(c * E + base, BLK)], sl_[9])

        def spill_wait(blk, sl_):
            base = s * EPT + blk * BLK
            pltpu.make_async_copy(sl_[8],
                                  spill_hbm.at[pl.ds(c * E + base, BLK)],
                                  sl_[9]).wait()

        def sc_wait(sl_):
            pltpu.make_async_copy(sl_[8], acc.at[sl_[4]], sl_[10]).wait()

        def step1(blk, cur, nxt, fire_next, fire_ids, drain_spill,
                  drain_sc):
            if drain_sc:
                sc_wait(nxt)
            gath_wait(blk, cur)
            if fire_next:
                ids_wait(blk + 1, nxt)
                prep(nxt)
                gath_fire(blk + 1, nxt)
            if fire_ids:
                ids_fire(blk + 2, cur)
            if drain_spill:
                spill_wait(blk - 2, cur)
            compute(cur)
            pltpu.async_copy(cur[8], acc.at[cur[4]], cur[10], add=True)
            spill_fire(blk, cur)

        ids_fire(0, slot0)
        ids_wait(0, slot0)
        prep(slot0)
        gath_fire(0, slot0)
        ids_fire(1, slot1)

        step1(0, slot0, slot1, True, True, False, False)
        step1(1, slot1, slot0, True, True, False, True)

        def pair1(t, _):
            b0 = 2 * t
            step1(b0, slot0, slot1, True, True, True, True)
            step1(b0 + 1, slot1, slot0, True, True, True, True)
            return 0

        lax.fori_loop(1, NBLK // 2 - 1, pair1, 0)
        step1(NBLK - 2, slot0, slot1, True, False, True, True)
        step1(NBLK - 1, slot1, slot0, False, False, True, True)
        spill_wait(NBLK - 2, slot0)
        spill_wait(NBLK - 1, slot1)
        sc_wait(slot1)

        plsc.subcore_barrier()
        for kk in range(-(-(N1 // FCH) // NS)):
            ch = s + kk * NS
            @pl.when(ch < N1 // FCH)
            def _():
                pltpu.sync_copy(acc.at[pl.ds(ch * FCH, FCH)],
                                out_hbm.at[pl.ds(coff + ch * FCH, FCH)])
        plsc.subcore_barrier()

        zero_acc()
        plsc.subcore_barrier()

        def rd_fire(blk, sl_):
            base = s * EPT + blk * BLK
            pltpu.async_copy(dst_hbm.at[pl.ds(base, BLK)], sl_[1], semi)
            pltpu.async_copy(spill_hbm.at[pl.ds(c * E + base, BLK)],
                             sl_[8], semg)

        def rd_wait(blk, sl_):
            base = s * EPT + blk * BLK
            pltpu.make_async_copy(dst_hbm.at[pl.ds(base, BLK)], sl_[1],
                                  semi).wait()
            pltpu.make_async_copy(spill_hbm.at[pl.ds(c * E + base, BLK)],
                                  sl_[8], semg).wait()

        def step2(blk, cur, nxt, fire_next):
            rd_wait(blk, cur)
            if fire_next:
                rd_fire(blk + 1, nxt)
            db, qi = cur[1], cur[4]
            for kk in range(BLK // L):
                sl = pl.ds(kk * L, L)
                dv = db[sl] - N1
                qi[sl] = jnp.where(dv >= 0, dv, N2)
            pltpu.sync_copy(cur[8], acc.at[qi], add=True)

        rd_fire(0, slot0)

        def pair2(t, _):
            step2(2 * t, slot0, slot1, True)
            step2(2 * t + 1, slot1, slot0, True)
            return 0

        lax.fori_loop(0, NBLK // 2 - 1, pair2, 0)
        step2(NBLK - 2, slot0, slot1, True)
        step2(NBLK - 1, slot1, slot0, False)

        plsc.subcore_barrier()
        for kk in range(-(-(N2 // FCH) // NS)):
            ch = s + kk * NS
            @pl.when(ch < N2 // FCH)
            def _():
                pltpu.sync_copy(acc.at[pl.ds(ch * FCH, FCH)],
                                out_hbm.at[pl.ds(coff + N1 + ch * FCH, FCH)])

    return k


def _epi_body(x_ref, a_ref, a0_ref, a1_ref, out_ref):
    xb = x_ref[...]
    a0 = a0_ref[...]
    a1 = a1_ref[...]
    num = jnp.concatenate([a0[:, :HALF], a1[:, :HALF]], axis=1)
    den = jnp.concatenate([a0[:, HALF:], a1[:, HALF:]], axis=1) + 1e-6
    h = jnp.dot(xb, a_ref[...], preferred_element_type=jnp.float32)
    h = h + num / den
    out_ref[...] = xb + jnp.maximum(h, 0.0)


def _epilogue(x, A, accd):
    N, d = x.shape
    NB = 2000
    nb = N // NB
    return pl.pallas_call(
        _epi_body,
        grid=(nb,),
        in_specs=[
            pl.BlockSpec((NB, d), lambda i: (i, 0)),
            pl.BlockSpec((d, d), lambda i: (0, 0)),
            pl.BlockSpec((NB, d), lambda i: (i, 0)),
            pl.BlockSpec((NB, d), lambda i, nb=nb: (i + nb, 0)),
        ],
        out_specs=pl.BlockSpec((NB, d), lambda i: (i, 0)),
        out_shape=jax.ShapeDtypeStruct((N, d), jnp.float32),
    )(x, A, accd, accd)


def kernel(x, edge_index, edge_attr, A, B, C, D, Ew):
    N, d = x.shape
    E = edge_index.shape[1]
    src = edge_index[0]
    dst = edge_index[1]

    tsrc, tdst = _make_tables(x, B, D, Ew)
    ec = _make_ec(edge_attr, C)
    tsrc = tsrc.reshape(NC * N, d)
    tdst = tdst.reshape(NC * N, d)
    ec = ec.reshape(NC * E, HALF)

    accd, _spill = _sc_gate(N, E, BLK=80)(src, dst, tsrc, tdst, ec)
    return _epilogue(x, A, accd)


# speedup vs baseline: 1.0156x; 1.0109x over previous
import functools

import jax
import jax.numpy as jnp
from jax import lax
from jax.experimental import pallas as pl
from jax.experimental.pallas import tpu as pltpu
from jax.experimental.pallas import tpu_sc as plsc

NC = 2
NS = 16
L = 16

HALF = 64


def _pre_body(x_ref, b_ref, d_ref, ew_ref, ea_ref, c_ref,
              tsrc_ref, tdst_ref, ec_ref):
    i = pl.program_id(0)

    @pl.when(i < 5)
    def _():
        xb = x_ref[...]
        xB = jnp.dot(xb, b_ref[...], preferred_element_type=jnp.float32)
        xD = jnp.dot(xb, d_ref[...], preferred_element_type=jnp.float32)
        xE = jnp.dot(xb, ew_ref[...], preferred_element_type=jnp.float32)
        tsrc_ref[0] = jnp.concatenate([xE[:, :HALF], xB[:, :HALF]], axis=1)
        tsrc_ref[1] = jnp.concatenate([xE[:, HALF:], xB[:, HALF:]], axis=1)
        z = jnp.zeros_like(xD[:, :HALF])
        tdst_ref[0] = jnp.concatenate([xD[:, :HALF], z], axis=1)
        tdst_ref[1] = jnp.concatenate([xD[:, HALF:], z], axis=1)

    @pl.when(i >= 5)
    def _():
        ev = jnp.dot(ea_ref[...], c_ref[...],
                     preferred_element_type=jnp.float32)
        ec_ref[0] = ev[:, :HALF]
        ec_ref[1] = ev[:, HALF:]


def _make_pre(x, edge_attr, B, D, Ew, C):
    N, d = x.shape
    E, de = edge_attr.shape
    NB = 2000
    EB = 8000
    nb = N // NB
    return pl.pallas_call(
        _pre_body,
        grid=(nb + E // EB,),
        in_specs=[
            pl.BlockSpec((NB, d), lambda i, nb=nb: (jnp.minimum(i, nb - 1), 0)),
            pl.BlockSpec((d, d), lambda i: (0, 0)),
            pl.BlockSpec((d, d), lambda i: (0, 0)),
            pl.BlockSpec((d, d), lambda i: (0, 0)),
            pl.BlockSpec((EB, de), lambda i, nb=nb: (jnp.maximum(i - nb, 0), 0)),
            pl.BlockSpec((de, d), lambda i: (0, 0)),
        ],
        out_specs=[
            pl.BlockSpec((2, NB, d), lambda i, nb=nb: (0, jnp.minimum(i, nb - 1), 0)),
            pl.BlockSpec((2, NB, d), lambda i, nb=nb: (0, jnp.minimum(i, nb - 1), 0)),
            pl.BlockSpec((2, EB, HALF), lambda i, nb=nb: (0, jnp.maximum(i - nb, 0), 0)),
        ],
        out_shape=[
            jax.ShapeDtypeStruct((2, N, d), jnp.float32),
            jax.ShapeDtypeStruct((2, N, d), jnp.float32),
            jax.ShapeDtypeStruct((2, E, HALF), jnp.float32),
        ],
    )(x, B, D, Ew, edge_attr, C)


def _sc_gate(N, E, BLK):
    EPT = E // NS
    NBLK = EPT // BLK
    d = 2 * HALF

    N1 = 5000
    N2 = N - N1
    AR = 5040
    ZCH = 48
    FCH = 40

    mesh = plsc.VectorSubcoreMesh(core_axis_name="c", subcore_axis_name="s")

    @functools.partial(
        pl.kernel,
        mesh=mesh,
        out_type=[
            jax.ShapeDtypeStruct((NC * N, d), jnp.float32),
            jax.ShapeDtypeStruct((NC * E, d), jnp.float32),
        ],
        scratch_types=(
            [pltpu.VMEM((BLK,), jnp.int32)] * 10 +
            [pltpu.VMEM((BLK, HALF), jnp.float32)] * 2 +
            [pltpu.VMEM((BLK, d), jnp.float32)] * 6 +
            [pltpu.VMEM((ZCH, d), jnp.float32),
             pltpu.VMEM_SHARED((AR, d), jnp.float32)] +
            [pltpu.SemaphoreType.DMA] * 4
        ),
    )
    def k(src_hbm, dst_hbm, tsrc_hbm, tdst_hbm, ec_hbm, out_hbm, spill_hbm,
          sidx0, didx0, gsidx0, gdidx0, qidx0,
          sidx1, didx1, gsidx1, gdidx1, qidx1,
          ecv0, ecv1, tsv0, tdv0, msd0, tsv1, tdv1, msd1, zbuf, acc,
          semi, semg, semsA, semsB):
        c = lax.axis_index("c")
        s = lax.axis_index("s")
        coff = c * N
        slot0 = (sidx0, didx0, gsidx0, gdidx0, qidx0, ecv0, tsv0, tdv0,
                 msd0, semsA)
        slot1 = (sidx1, didx1, gsidx1, gdidx1, qidx1, ecv1, tsv1, tdv1,
                 msd1, semsB)

        zv = jnp.zeros((L,), jnp.float32)

        def zloop(t, _):
            i = t // (d // L)
            j = t % (d // L)
            zbuf[i, pl.ds(j * L, L)] = zv
            return 0

        lax.fori_loop(0, ZCH * (d // L), zloop, 0)

        def zero_acc():
            for kk in range(-(-(AR // ZCH) // NS)):
                ch = s + kk * NS
                @pl.when(ch < AR // ZCH)
                def _():
                    pltpu.sync_copy(zbuf, acc.at[pl.ds(ch * ZCH, ZCH)])

        zero_acc()
        plsc.subcore_barrier()

        def ids_fire(blk, sl_):
            base = s * EPT + blk * BLK
            pltpu.async_copy(src_hbm.at[pl.ds(base, BLK)], sl_[0], semi)
            pltpu.async_copy(dst_hbm.at[pl.ds(base, BLK)], sl_[1], semi)

        def ids_wait(blk, sl_):
            base = s * EPT + blk * BLK
            pltpu.make_async_copy(src_hbm.at[pl.ds(base, BLK)], sl_[0],
                                  semi).wait()
            pltpu.make_async_copy(dst_hbm.at[pl.ds(base, BLK)], sl_[1],
                                  semi).wait()

        def prep(sl_):
            sb, db, gs, gd, qi = sl_[0], sl_[1], sl_[2], sl_[3], sl_[4]
            for kk in range(BLK // L):
                sl = pl.ds(kk * L, L)
                dv = db[sl]
                gs[sl] = sb[sl] + coff
                gd[sl] = dv + coff
                qi[sl] = jnp.where(dv < N1, dv, N1)

        def gath_fire(blk, sl_):
            base = s * EPT + blk * BLK
            pltpu.async_copy(tsrc_hbm.at[sl_[2]], sl_[6], semg)
            pltpu.async_copy(tdst_hbm.at[sl_[3]], sl_[7], semg)
            pltpu.async_copy(ec_hbm.at[pl.ds(c * E + base, BLK)], sl_[5],
                             semg)

        def gath_wait(blk, sl_):
            base = s * EPT + blk * BLK
            pltpu.make_async_copy(tsrc_hbm.at[sl_[2]], sl_[6], semg).wait()
            pltpu.make_async_copy(tdst_hbm.at[sl_[3]], sl_[7], semg).wait()
            pltpu.make_async_copy(ec_hbm.at[pl.ds(c * E + base, BLK)],
                                  sl_[5], semg).wait()

        def compute(sl_):
            ecb, tsb, tdb, msb = sl_[5], sl_[6], sl_[7], sl_[8]

            def edge(j, _):
                for q in range(HALF // L):
                    sl = pl.ds(q * L, L)
                    sh = pl.ds(HALF + q * L, L)
                    e = ecb[j, sl] + tdb[j, sl] + tsb[j, sl]
                    sg = 1.0 / (1.0 + jnp.exp(-e))
                    msb[j, sl] = sg * tsb[j, sh]
                    msb[j, sh] = sg
                return 0

            lax.fori_loop(0, BLK, edge, 0)

        def spill_fire(blk, sl_):
            base = s * EPT + blk * BLK
            pltpu.async_copy(sl_[8],
                             spill_hbm.at[pl.ds(c * E + base, BLK)], sl_[9])

        def spill_wait(blk, sl_):
            base = s * EPT + blk * BLK
            pltpu.make_async_copy(sl_[8],
                                  spill_hbm.at[pl.ds(c * E + base, BLK)],
                                  sl_[9]).wait()

        def step1(blk, cur, nxt, fire_next, fire_ids, drain_spill):
            gath_wait(blk, cur)
            if fire_next:
                ids_wait(blk + 1, nxt)
                prep(nxt)
                gath_fire(blk + 1, nxt)
            if fire_ids:
                ids_fire(blk + 2, cur)
            if drain_spill:
                spill_wait(blk - 2, cur)
            compute(cur)
            pltpu.sync_copy(cur[8], acc.at[cur[4]], add=True)
            spill_fire(blk, cur)

        ids_fire(0, slot0)
        ids_wait(0, slot0)
        prep(slot0)
        gath_fire(0, slot0)
        ids_fire(1, slot1)

        step1(0, slot0, slot1, True, True, False)
        step1(1, slot1, slot0, True, True, False)

        def pair1(t, _):
            b0 = 2 * t
            step1(b0, slot0, slot1, True, True, True)
            step1(b0 + 1, slot1, slot0, True, True, True)
            return 0

        lax.fori_loop(1, NBLK // 2 - 1, pair1, 0)
        step1(NBLK - 2, slot0, slot1, True, False, True)
        step1(NBLK - 1, slot1, slot0, False, False, True)
        spill_wait(NBLK - 2, slot0)
        spill_wait(NBLK - 1, slot1)

        plsc.subcore_barrier()
        for kk in range(-(-(N1 // FCH) // NS)):
            ch = s + kk * NS
            @pl.when(ch < N1 // FCH)
            def _():
                pltpu.sync_copy(acc.at[pl.ds(ch * FCH, FCH)],
                                out_hbm.at[pl.ds(coff + ch * FCH, FCH)])
        plsc.subcore_barrier()

        zero_acc()
        plsc.subcore_barrier()

        def rd_fire(blk, sl_):
            base = s * EPT + blk * BLK
            pltpu.async_copy(dst_hbm.at[pl.ds(base, BLK)], sl_[1], semi)
            pltpu.async_copy(spill_hbm.at[pl.ds(c * E + base, BLK)],
                             sl_[8], semg)

        def rd_wait(blk, sl_):
            base = s * EPT + blk * BLK
            pltpu.make_async_copy(dst_hbm.at[pl.ds(base, BLK)], sl_[1],
                                  semi).wait()
            pltpu.make_async_copy(spill_hbm.at[pl.ds(c * E + base, BLK)],
                                  sl_[8], semg).wait()

        def step2(blk, cur, nxt, fire_next):
            rd_wait(blk, cur)
            if fire_next:
                rd_fire(blk + 1, nxt)
            db, qi = cur[1], cur[4]
            for kk in range(BLK // L):
                sl = pl.ds(kk * L, L)
                dv = db[sl] - N1
                qi[sl] = jnp.where(dv >= 0, dv, N2)
            pltpu.sync_copy(cur[8], acc.at[qi], add=True)

        rd_fire(0, slot0)

        def pair2(t, _):
            step2(2 * t, slot0, slot1, True)
            step2(2 * t + 1, slot1, slot0, True)
            return 0

        lax.fori_loop(0, NBLK // 2 - 1, pair2, 0)
        step2(NBLK - 2, slot0, slot1, True)
        step2(NBLK - 1, slot1, slot0, False)

        plsc.subcore_barrier()
        for kk in range(-(-(N2 // FCH) // NS)):
            ch = s + kk * NS
            @pl.when(ch < N2 // FCH)
            def _():
                pltpu.sync_copy(acc.at[pl.ds(ch * FCH, FCH)],
                                out_hbm.at[pl.ds(coff + N1 + ch * FCH, FCH)])

    return k


def _epi_body(x_ref, a_ref, a0_ref, a1_ref, out_ref):
    xb = x_ref[...]
    a0 = a0_ref[...]
    a1 = a1_ref[...]
    num = jnp.concatenate([a0[:, :HALF], a1[:, :HALF]], axis=1)
    den = jnp.concatenate([a0[:, HALF:], a1[:, HALF:]], axis=1) + 1e-6
    h = jnp.dot(xb, a_ref[...], preferred_element_type=jnp.float32)
    h = h + num / den
    out_ref[...] = xb + jnp.maximum(h, 0.0)


def _epilogue(x, A, accd):
    N, d = x.shape
    NB = 2000
    nb = N // NB
    return pl.pallas_call(
        _epi_body,
        grid=(nb,),
        in_specs=[
            pl.BlockSpec((NB, d), lambda i: (i, 0)),
            pl.BlockSpec((d, d), lambda i: (0, 0)),
            pl.BlockSpec((NB, d), lambda i: (i, 0)),
            pl.BlockSpec((NB, d), lambda i, nb=nb: (i + nb, 0)),
        ],
        out_specs=pl.BlockSpec((NB, d), lambda i: (i, 0)),
        out_shape=jax.ShapeDtypeStruct((N, d), jnp.float32),
    )(x, A, accd, accd)


def kernel(x, edge_index, edge_attr, A, B, C, D, Ew):
    N, d = x.shape
    E = edge_index.shape[1]
    src = edge_index[0]
    dst = edge_index[1]

    tsrc, tdst, ec = _make_pre(x, edge_attr, B, D, Ew, C)
    tsrc = tsrc.reshape(NC * N, d)
    tdst = tdst.reshape(NC * N, d)
    ec = ec.reshape(NC * E, HALF)

    accd, _spill = _sc_gate(N, E, BLK=80)(src, dst, tsrc, tdst, ec)
    return _epilogue(x, A, accd)
